# baseline (speedup 1.0000x reference)
import functools

import jax
import jax.numpy as jnp
from jax import lax
from jax.experimental import pallas as pl
from jax.experimental.pallas import tpu as pltpu
from jax.experimental.pallas import tpu_sc as plsc

VOCAB = 1000000
D = 64
BATCH = 16384
HIST = 200

NC = 2
NS = 16
NW = NC * NS

BAGS_PER_W = BATCH // NW
CHUNK = 4
ROWS = CHUNK * HIST
NCHUNK = BAGS_PER_W // CHUNK
V = 16
TR = 80
NTR = ROWS // TR
UNR = 8


def _emb_bag_body(ids, table, out,
                  idx_v, rows_v, o0_v, o1_v,
                  gs0, gs1, ws0, ws1):
    sid = lax.axis_index("s")
    wid = sid * NC + lax.axis_index("c")
    w_bag0 = wid * BAGS_PER_W

    gsems = (gs0, gs1)
    wsems = (ws0, ws1)
    outvs = (o0_v, o1_v)

    def stage_and_gather(g, slot):
        pltpu.sync_copy(ids.at[pl.ds((w_bag0 + g * CHUNK) * HIST // TR, NTR)],
                        idx_v.at[slot])
        for t in range(NTR):
            pltpu.async_copy(table.at[idx_v.at[slot, t]],
                             rows_v.at[slot, pl.ds(t * TR, TR)],
                             gsems[slot])

    def drain_gather(slot):
        for t in range(NTR):
            pltpu.make_async_copy(table.at[idx_v.at[slot, t]],
                                  rows_v.at[slot, pl.ds(t * TR, TR)],
                                  gsems[slot]).wait()

    def reduce_chunk(g, slot):
        ov = outvs[slot]
        z = jnp.zeros((V,), jnp.float32)
        mask = jnp.full((V,), -65536, jnp.int32)

        def unpack_lo(v):
            return lax.bitcast_convert_type(
                lax.shift_left(v, jnp.full((V,), 16, jnp.int32)),
                jnp.float32)

        def unpack_hi(v):
            return lax.bitcast_convert_type(
                lax.bitwise_and(v, mask), jnp.float32)

        for i in range(CHUNK):
            def body(jb, accs, base=i * HIST):
                a0, a1, a2, a3 = accs
                for jj in range(UNR):
                    r = base + jb * UNR + jj
                    v0 = rows_v[slot, r, pl.ds(0, V)]
                    v1 = rows_v[slot, r, pl.ds(V, V)]
                    a0 = a0 + unpack_lo(v0)
                    a1 = a1 + unpack_hi(v0)
                    a2 = a2 + unpack_lo(v1)
                    a3 = a3 + unpack_hi(v1)
                return (a0, a1, a2, a3)

            accs = lax.fori_loop(0, HIST // UNR, body, (z,) * 4)
            for k in range(D // V):
                ov[i, pl.ds(k * V, V)] = accs[k]
        pltpu.async_copy(ov,
                         out.at[pl.ds(w_bag0 + g * CHUNK, CHUNK)],
                         wsems[slot])

    stage_and_gather(0, 0)
    stage_and_gather(1, 1)

    def pair(p, carry):
        g0 = 2 * p

        drain_gather(0)

        @pl.when(p > 0)
        def _():
            pltpu.make_async_copy(
                o0_v, out.at[pl.ds(w_bag0, CHUNK)], ws0).wait()

        reduce_chunk(g0, 0)

        @pl.when(g0 + 2 < NCHUNK)
        def _():
            stage_and_gather(g0 + 2, 0)

        drain_gather(1)

        @pl.when(p > 0)
        def _():
            pltpu.make_async_copy(
                o1_v, out.at[pl.ds(w_bag0, CHUNK)], ws1).wait()

        reduce_chunk(g0 + 1, 1)

        @pl.when(g0 + 3 < NCHUNK)
        def _():
            stage_and_gather(g0 + 3, 1)

        return carry

    lax.fori_loop(0, NCHUNK // 2, pair, 0)

    pltpu.make_async_copy(o0_v, out.at[pl.ds(w_bag0, CHUNK)], ws0).wait()
    pltpu.make_async_copy(o1_v, out.at[pl.ds(w_bag0, CHUNK)], ws1).wait()


@jax.jit
def _emb_bag(ids_flat, table):
    mesh = plsc.VectorSubcoreMesh(core_axis_name="c", subcore_axis_name="s")
    return pl.kernel(
        _emb_bag_body,
        out_type=jax.ShapeDtypeStruct((BATCH, D), jnp.float32),
        mesh=mesh,
        scratch_types=[
            pltpu.VMEM((2, NTR, TR), jnp.int32),
            pltpu.VMEM((2, ROWS, D // 2), jnp.int32),
            pltpu.VMEM((CHUNK, D), jnp.float32),
            pltpu.VMEM((CHUNK, D), jnp.float32),
            pltpu.SemaphoreType.DMA,
            pltpu.SemaphoreType.DMA,
            pltpu.SemaphoreType.DMA,
            pltpu.SemaphoreType.DMA,
        ],
        compiler_params=pltpu.CompilerParams(use_tc_tiling_on_sc=False),
    )(ids_flat, table)


def _mlp_body(x_ref, w1_ref, b1_ref, w2_ref, b2_ref, wc_ref, bc_ref, o_ref):
    x = x_ref[...]
    w1 = w1_ref[...] * jnp.float32(1.0 / HIST)
    h = jnp.maximum(
        jnp.dot(x, w1, preferred_element_type=jnp.float32)
        + b1_ref[...], 0.0)
    h = jnp.maximum(
        jnp.dot(h, w2_ref[...], preferred_element_type=jnp.float32)
        + b2_ref[...], 0.0)
    o_ref[...] = (jnp.dot(h, wc_ref[...], preferred_element_type=jnp.float32)
                  + bc_ref[...])


@jax.jit
def _mlp(x, W1, b1, W2, b2, Wc, bc):
    BM = 2048
    nb = BATCH // BM
    rep = lambda i: (0, 0)
    return pl.pallas_call(
        _mlp_body,
        grid=(nb,),
        in_specs=[
            pl.BlockSpec((BM, D), lambda i: (i, 0)),
            pl.BlockSpec(W1.shape, rep),
            pl.BlockSpec(b1.shape, rep),
            pl.BlockSpec(W2.shape, rep),
            pl.BlockSpec(b2.shape, rep),
            pl.BlockSpec(Wc.shape, rep),
            pl.BlockSpec(bc.shape, rep),
        ],
        out_specs=pl.BlockSpec((BM, 2), lambda i: (i, 0)),
        out_shape=jax.ShapeDtypeStruct((BATCH, 2), jnp.float32),
    )(x, W1, b1, W2, b2, Wc, bc)


def kernel(input_ids, emb_table, W1, b1, W2, b2, Wc, bc):
    ids_flat = input_ids.reshape(BATCH * HIST // TR, TR)
    c = jnp.arange(D)
    col = (c // 32) * 32 + jnp.where(c % 2 == 0, (c % 32) // 2,
                                     16 + (c % 32) // 2)
    t16 = emb_table.astype(jnp.bfloat16)[:, col]
    packed = lax.bitcast_convert_type(t16.reshape(VOCAB, D // 2, 2),
                                      jnp.int32)
    summed = _emb_bag(ids_flat, packed)
    return _mlp(summed, W1, b1.reshape(1, -1), W2, b2.reshape(1, -1),
                Wc, bc.reshape(1, -1))


# speedup vs baseline: 2.6497x; 2.6497x over previous
import functools

import jax
import jax.numpy as jnp
from jax import lax
from jax.experimental import pallas as pl
from jax.experimental.pallas import tpu as pltpu
from jax.experimental.pallas import tpu_sc as plsc

VOCAB = 1000000
D = 64
BATCH = 16384
HIST = 200

NC = 2
NS = 16
NW = NC * NS

BAGS_PER_W = BATCH // NW
CHUNK = 4
ROWS = CHUNK * HIST
NCHUNK = BAGS_PER_W // CHUNK
V = 16
TR = 80
NTR = ROWS // TR
UNR = 8


def _emb_bag_body(ids, table, out,
                  idx_v, rows_v, o0_v, o1_v,
                  gs0, gs1, ws0, ws1, is0, is1):
    sid = lax.axis_index("s")
    wid = sid * NC + lax.axis_index("c")
    w_bag0 = wid * BAGS_PER_W

    gsems = (gs0, gs1)
    wsems = (ws0, ws1)
    isems = (is0, is1)
    outvs = (o0_v, o1_v)

    def stage_async(g, slot):
        pltpu.async_copy(ids.at[pl.ds((w_bag0 + g * CHUNK) * HIST // TR, NTR)],
                         idx_v.at[slot], isems[slot])

    def fire_gather(g, slot):
        pltpu.make_async_copy(
            ids.at[pl.ds((w_bag0 + g * CHUNK) * HIST // TR, NTR)],
            idx_v.at[slot], isems[slot]).wait()
        for t in range(NTR):
            pltpu.async_copy(table.at[idx_v.at[slot, t]],
                             rows_v.at[slot, pl.ds(t * TR, TR)],
                             gsems[slot])

    def drain_gather(slot):
        for t in range(NTR):
            pltpu.make_async_copy(table.at[idx_v.at[slot, t]],
                                  rows_v.at[slot, pl.ds(t * TR, TR)],
                                  gsems[slot]).wait()

    def reduce_chunk(g, slot):
        ov = outvs[slot]
        z = jnp.zeros((V,), jnp.float32)

        for i in range(CHUNK):
            def body(jb, accs, base=i * HIST):
                new = []
                for k in range(D // V):
                    a = accs[k]
                    for jj in range(UNR):
                        r = base + jb * UNR + jj
                        a = a + rows_v[slot, r, pl.ds(k * V, V)]
                    new.append(a)
                return tuple(new)

            accs = lax.fori_loop(0, HIST // UNR, body, (z,) * (D // V))
            for k in range(D // V):
                ov[i, pl.ds(k * V, V)] = accs[k]
        pltpu.async_copy(ov,
                         out.at[pl.ds(w_bag0 + g * CHUNK, CHUNK)],
                         wsems[slot])

    stage_async(0, 0)
    stage_async(1, 1)
    fire_gather(0, 0)
    fire_gather(1, 1)

    def pair(p, carry):
        g0 = 2 * p

        drain_gather(0)

        @pl.when(g0 + 2 < NCHUNK)
        def _():
            stage_async(g0 + 2, 0)

        @pl.when(p > 0)
        def _():
            pltpu.make_async_copy(
                o0_v, out.at[pl.ds(w_bag0, CHUNK)], ws0).wait()

        reduce_chunk(g0, 0)

        @pl.when(g0 + 2 < NCHUNK)
        def _():
            fire_gather(g0 + 2, 0)

        drain_gather(1)

        @pl.when(g0 + 3 < NCHUNK)
        def _():
            stage_async(g0 + 3, 1)

        @pl.when(p > 0)
        def _():
            pltpu.make_async_copy(
                o1_v, out.at[pl.ds(w_bag0, CHUNK)], ws1).wait()

        reduce_chunk(g0 + 1, 1)

        @pl.when(g0 + 3 < NCHUNK)
        def _():
            fire_gather(g0 + 3, 1)

        return carry

    lax.fori_loop(0, NCHUNK // 2, pair, 0)

    pltpu.make_async_copy(o0_v, out.at[pl.ds(w_bag0, CHUNK)], ws0).wait()
    pltpu.make_async_copy(o1_v, out.at[pl.ds(w_bag0, CHUNK)], ws1).wait()


@jax.jit
def _emb_bag(ids_flat, table):
    mesh = plsc.VectorSubcoreMesh(core_axis_name="c", subcore_axis_name="s")
    return pl.kernel(
        _emb_bag_body,
        out_type=jax.ShapeDtypeStruct((BATCH, D), jnp.float32),
        mesh=mesh,
        scratch_types=[
            pltpu.VMEM((2, NTR, TR), jnp.int32),
            pltpu.VMEM((2, ROWS, D), jnp.float32),
            pltpu.VMEM((CHUNK, D), jnp.float32),
            pltpu.VMEM((CHUNK, D), jnp.float32),
            pltpu.SemaphoreType.DMA,
            pltpu.SemaphoreType.DMA,
            pltpu.SemaphoreType.DMA,
            pltpu.SemaphoreType.DMA,
            pltpu.SemaphoreType.DMA,
            pltpu.SemaphoreType.DMA,
        ],
        compiler_params=pltpu.CompilerParams(use_tc_tiling_on_sc=False),
    )(ids_flat, table)


def _mlp_body(x_ref, w1_ref, b1_ref, w2_ref, b2_ref, wc_ref, bc_ref, o_ref):
    x = x_ref[...]
    w1 = w1_ref[...] * jnp.float32(1.0 / HIST)
    h = jnp.maximum(
        jnp.dot(x, w1, preferred_element_type=jnp.float32)
        + b1_ref[...], 0.0)
    h = jnp.maximum(
        jnp.dot(h, w2_ref[...], preferred_element_type=jnp.float32)
        + b2_ref[...], 0.0)
    o_ref[...] = (jnp.dot(h, wc_ref[...], preferred_element_type=jnp.float32)
                  + bc_ref[...])


@jax.jit
def _mlp(x, W1, b1, W2, b2, Wc, bc):
    BM = 2048
    nb = BATCH // BM
    rep = lambda i: (0, 0)
    return pl.pallas_call(
        _mlp_body,
        grid=(nb,),
        in_specs=[
            pl.BlockSpec((BM, D), lambda i: (i, 0)),
            pl.BlockSpec(W1.shape, rep),
            pl.BlockSpec(b1.shape, rep),
            pl.BlockSpec(W2.shape, rep),
            pl.BlockSpec(b2.shape, rep),
            pl.BlockSpec(Wc.shape, rep),
            pl.BlockSpec(bc.shape, rep),
        ],
        out_specs=pl.BlockSpec((BM, 2), lambda i: (i, 0)),
        out_shape=jax.ShapeDtypeStruct((BATCH, 2), jnp.float32),
    )(x, W1, b1, W2, b2, Wc, bc)


def kernel(input_ids, emb_table, W1, b1, W2, b2, Wc, bc):
    ids_flat = input_ids.reshape(BATCH * HIST // TR, TR)
    summed = _emb_bag(ids_flat, emb_table)
    return _mlp(summed, W1, b1.reshape(1, -1), W2, b2.reshape(1, -1),
                Wc, bc.reshape(1, -1))
